# baseline (speedup 1.0000x reference)
import jax
import jax.numpy as jnp
from jax.experimental import pallas as pl


_LOG2 = 0.6931471805599453


def _ssp(x):
    return jnp.maximum(x, 0.0) + jnp.log1p(jnp.exp(-jnp.abs(x))) - _LOG2


def _filters_body(e_ref, w1_ref, b1_ref, w2_ref, b2_ref, o_ref):
    h = jnp.dot(e_ref[...], w1_ref[...], preferred_element_type=jnp.float32)
    h = _ssp(h + b1_ref[...])
    h = jnp.dot(h, w2_ref[...], preferred_element_type=jnp.float32)
    o_ref[...] = _ssp(h + b2_ref[...])


def _compute_filters(edges, W1, b1, W2, b2):
    E, DE = edges.shape
    D = W1.shape[1]
    BE = 512
    grid = E // BE
    return pl.pallas_call(
        _filters_body,
        grid=(grid,),
        in_specs=[
            pl.BlockSpec((BE, DE), lambda i: (i, 0)),
            pl.BlockSpec((DE, D), lambda i: (0, 0)),
            pl.BlockSpec((1, D), lambda i: (0, 0)),
            pl.BlockSpec((D, D), lambda i: (0, 0)),
            pl.BlockSpec((1, D), lambda i: (0, 0)),
        ],
        out_specs=pl.BlockSpec((BE, D), lambda i: (i, 0)),
        out_shape=jax.ShapeDtypeStruct((E, D), jnp.float32),
    )(edges, W1, b1.reshape(1, D), W2, b2.reshape(1, D))


def kernel(nodes, edges, edges_i, W1, b1, W2, b2):
    filters = _compute_filters(edges, W1, b1, W2, b2)
    starts = edges_i[:, 0]
    ends = edges_i[:, 1]
    conv = jnp.take(nodes, ends, axis=0) * filters
    return nodes.at[starts].add(conv)


# speedup vs baseline: 3.5672x; 3.5672x over previous
import functools

import jax
import jax.numpy as jnp
from jax import lax
from jax.experimental import pallas as pl
from jax.experimental.pallas import tpu as pltpu
from jax.experimental.pallas import tpu_sc as plsc

_LOG2 = 0.6931471805599453
_K = 80


def _ssp(x):
    return jnp.maximum(x, 0.0) + jnp.log1p(jnp.exp(-jnp.abs(x))) - _LOG2


def _filters_body(e_ref, w1_ref, b1_ref, w2_ref, b2_ref, o_ref):
    h = jnp.dot(e_ref[...], w1_ref[...], preferred_element_type=jnp.float32)
    h = _ssp(h + b1_ref[...])
    h = jnp.dot(h, w2_ref[...], preferred_element_type=jnp.float32)
    o_ref[...] = _ssp(h + b2_ref[...])


def _compute_filters(edges, W1, b1, W2, b2):
    E, DE = edges.shape
    D = W1.shape[1]
    BE = 2000
    grid = E // BE
    return pl.pallas_call(
        _filters_body,
        grid=(grid,),
        in_specs=[
            pl.BlockSpec((BE, DE), lambda i: (i, 0)),
            pl.BlockSpec((DE, D), lambda i: (0, 0)),
            pl.BlockSpec((1, D), lambda i: (0, 0)),
            pl.BlockSpec((D, D), lambda i: (0, 0)),
            pl.BlockSpec((1, D), lambda i: (0, 0)),
        ],
        out_specs=pl.BlockSpec((BE, D), lambda i: (i, 0)),
        out_shape=jax.ShapeDtypeStruct((E, D), jnp.float32),
    )(edges, W1, b1.reshape(1, D), W2, b2.reshape(1, D))


def _sc_conv(nodes, filters, starts2, ends2):
    N, D = nodes.shape
    nchunks, k = starts2.shape
    info = plsc.get_sparse_core_info()
    NC, NS = info.num_cores, info.num_subcores
    NW = NC * NS
    CH = nchunks // NW
    RPS = (N // NS) // 8 * 8
    REM = N - NS * RPS
    assert nchunks % NW == 0 and k == _K and REM % 8 == 0

    starts3 = starts2.reshape(NW, CH, k)
    ends3 = ends2.reshape(NW, CH, k)

    mesh = plsc.VectorSubcoreMesh(
        core_axis_name="c", subcore_axis_name="s",
        num_cores=NC, num_subcores=NS)

    @functools.partial(
        pl.kernel,
        out_type=jax.ShapeDtypeStruct((NC, N, D), jnp.float32),
        mesh=mesh,
        scratch_types=[
            pltpu.VMEM((CH, _K), jnp.int32),
            pltpu.VMEM((2, _K), jnp.int32),
            pltpu.VMEM((_K, D), jnp.float32),
            pltpu.VMEM((_K, D), jnp.float32),
            pltpu.VMEM_SHARED((N, D), jnp.float32),
            pltpu.SemaphoreType.DMA,
            pltpu.SemaphoreType.DMA,
            pltpu.SemaphoreType.DMA,
        ],
    )
    def conv(nodes_hbm, filt_hbm, starts_hbm, ends_hbm, part_hbm,
             sidx_v, eidx_v, rows_v, filtc_v, acc, sem_g, sem_f, sem_e):
        c = lax.axis_index("c")
        s = lax.axis_index("s")
        wid = s * NC + c
        r0 = s * RPS
        pltpu.sync_copy(nodes_hbm.at[pl.ds(r0, RPS)], acc.at[pl.ds(r0, RPS)])

        @pl.when(s == 0)
        def _init_rem():
            pltpu.sync_copy(nodes_hbm.at[pl.ds(NS * RPS, REM)],
                            acc.at[pl.ds(NS * RPS, REM)])
        cbase = wid * CH
        pltpu.sync_copy(starts_hbm.at[wid], sidx_v)
        pltpu.async_copy(ends_hbm.at[wid, 0], eidx_v.at[0], sem_e)
        plsc.subcore_barrier()

        def chunk(j, carry):
            slot = lax.rem(j, 2)
            off = (cbase + j) * _K
            cp_f = pltpu.async_copy(filt_hbm.at[pl.ds(off, _K)], filtc_v, sem_f)
            pltpu.make_async_copy(ends_hbm.at[wid, j], eidx_v.at[slot],
                                  sem_e).wait()

            @pl.when(j < CH - 1)
            def _prefetch_next():
                pltpu.async_copy(ends_hbm.at[wid, j + 1], eidx_v.at[1 - slot],
                                 sem_e)

            cp_g = pltpu.async_copy(nodes_hbm.at[eidx_v.at[slot]], rows_v,
                                    sem_g)
            cp_f.wait()
            cp_g.wait()

            def mul_row(r, carry2):
                for col in range(D // 16):
                    sl = (r, pl.ds(col * 16, 16))
                    rows_v[sl] = rows_v[sl] * filtc_v[sl]
                return carry2

            lax.fori_loop(0, _K, mul_row, 0)
            pltpu.sync_copy(rows_v, acc.at[sidx_v.at[j]], add=True)
            return carry

        lax.fori_loop(0, CH, chunk, 0)

        plsc.subcore_barrier()
        pltpu.sync_copy(acc.at[pl.ds(r0, RPS)], part_hbm.at[c, pl.ds(r0, RPS)])

        @pl.when(s == 0)
        def _out_rem():
            pltpu.sync_copy(acc.at[pl.ds(NS * RPS, REM)],
                            part_hbm.at[c, pl.ds(NS * RPS, REM)])

    return conv(nodes, filters, starts3, ends3)


def _combine_body(p_ref, n_ref, o_ref):
    o_ref[...] = p_ref[0] + p_ref[1] - n_ref[...]


def _combine(part, nodes):
    N, D = nodes.shape
    BN = 1000
    grid = N // BN
    return pl.pallas_call(
        _combine_body,
        grid=(grid,),
        in_specs=[
            pl.BlockSpec((2, BN, D), lambda i: (0, i, 0)),
            pl.BlockSpec((BN, D), lambda i: (i, 0)),
        ],
        out_specs=pl.BlockSpec((BN, D), lambda i: (i, 0)),
        out_shape=jax.ShapeDtypeStruct((N, D), jnp.float32),
    )(part, nodes)


def kernel(nodes, edges, edges_i, W1, b1, W2, b2):
    filters = _compute_filters(edges, W1, b1, W2, b2)
    starts2 = edges_i[:, 0].reshape(-1, _K)
    ends2 = edges_i[:, 1].reshape(-1, _K)
    part = _sc_conv(nodes, filters, starts2, ends2)
    return _combine(part, nodes)


# speedup vs baseline: 4.1395x; 1.1604x over previous
import functools

import jax
import jax.numpy as jnp
from jax import lax
from jax.experimental import pallas as pl
from jax.experimental.pallas import tpu as pltpu
from jax.experimental.pallas import tpu_sc as plsc

_LOG2 = 0.6931471805599453
_K = 80
_BE = 2000
_SEGF = (0.2, 0.4, 0.4)


def _ssp(x):
    return jnp.maximum(x, 0.0) + jnp.log1p(jnp.exp(-jnp.abs(x))) - _LOG2


def _filters_body(e_ref, w1_ref, b1_ref, w2_ref, b2_ref, o_ref):
    h = jnp.dot(e_ref[...], w1_ref[...], preferred_element_type=jnp.float32)
    h = _ssp(h + b1_ref[...])
    h = jnp.dot(h, w2_ref[...], preferred_element_type=jnp.float32)
    o_ref[...] = _ssp(h + b2_ref[...])


def _compute_filters_seg(edges, W1, b1, W2, b2, row0, nrows):
    E, DE = edges.shape
    D = W1.shape[1]
    b0 = row0 // _BE
    grid = nrows // _BE
    return pl.pallas_call(
        _filters_body,
        grid=(grid,),
        in_specs=[
            pl.BlockSpec((_BE, DE), lambda i: (b0 + i, 0)),
            pl.BlockSpec((DE, D), lambda i: (0, 0)),
            pl.BlockSpec((1, D), lambda i: (0, 0)),
            pl.BlockSpec((D, D), lambda i: (0, 0)),
            pl.BlockSpec((1, D), lambda i: (0, 0)),
        ],
        out_specs=pl.BlockSpec((_BE, D), lambda i: (i, 0)),
        out_shape=jax.ShapeDtypeStruct((nrows, D), jnp.float32),
    )(edges, W1, b1.reshape(1, D), W2, b2.reshape(1, D))


def _sc_conv_seg(nodes, filters, starts3, ends3):
    N, D = nodes.shape
    info = plsc.get_sparse_core_info()
    NC, NS = info.num_cores, info.num_subcores
    NW = NC * NS
    chs = starts3.shape[1]
    RPS = (N // NS) // 8 * 8
    REM = N - NS * RPS
    ZCH = -(-RPS // _K)
    assert REM % 8 == 0 and starts3.shape[0] == NW

    mesh = plsc.VectorSubcoreMesh(
        core_axis_name="c", subcore_axis_name="s",
        num_cores=NC, num_subcores=NS)

    @functools.partial(
        pl.kernel,
        out_type=jax.ShapeDtypeStruct((NC, N, D), jnp.float32),
        mesh=mesh,
        scratch_types=[
            pltpu.VMEM((chs, _K), jnp.int32),
            pltpu.VMEM((2, _K), jnp.int32),
            pltpu.VMEM((_K, D), jnp.float32),
            pltpu.VMEM((_K, D), jnp.float32),
            pltpu.VMEM_SHARED((N, D), jnp.float32),
            pltpu.SemaphoreType.DMA,
            pltpu.SemaphoreType.DMA,
            pltpu.SemaphoreType.DMA,
        ],
    )
    def conv(nodes_hbm, filt_hbm, starts_hbm, ends_hbm, part_hbm,
             sidx_v, eidx_v, rows_v, filtc_v, acc, sem_g, sem_f, sem_e):
        c = lax.axis_index("c")
        s = lax.axis_index("s")
        wid = s * NC + c
        r0 = s * RPS

        zero16 = jnp.zeros((16,), jnp.float32)

        def zrow(r, carry):
            for col in range(D // 16):
                rows_v[r, pl.ds(col * 16, 16)] = zero16
            return carry

        lax.fori_loop(0, _K, zrow, 0)
        zcps = []
        for z in range(ZCH):
            rz = min(_K, RPS - z * _K)
            zcps.append(pltpu.async_copy(
                rows_v.at[pl.ds(0, rz)],
                acc.at[pl.ds(r0 + z * _K, rz)], sem_g))

        @pl.when(s == 0)
        def _zero_rem():
            pltpu.sync_copy(rows_v.at[pl.ds(0, REM)],
                            acc.at[pl.ds(NS * RPS, REM)])

        pltpu.sync_copy(starts_hbm.at[wid], sidx_v)
        pltpu.async_copy(ends_hbm.at[wid, 0], eidx_v.at[0], sem_e)
        for cp in zcps:
            cp.wait()
        plsc.subcore_barrier()

        def chunk(j, carry):
            slot = lax.rem(j, 2)
            off = (wid * chs + j) * _K
            cp_f = pltpu.async_copy(filt_hbm.at[pl.ds(off, _K)], filtc_v,
                                    sem_f)
            pltpu.make_async_copy(ends_hbm.at[wid, j], eidx_v.at[slot],
                                  sem_e).wait()

            @pl.when(j < chs - 1)
            def _prefetch_next():
                pltpu.async_copy(ends_hbm.at[wid, j + 1],
                                 eidx_v.at[1 - slot], sem_e)

            cp_g = pltpu.async_copy(nodes_hbm.at[eidx_v.at[slot]], rows_v,
                                    sem_g)
            cp_f.wait()
            cp_g.wait()

            def mul_row(r, carry2):
                for col in range(D // 16):
                    sl = (r, pl.ds(col * 16, 16))
                    rows_v[sl] = rows_v[sl] * filtc_v[sl]
                return carry2

            lax.fori_loop(0, _K, mul_row, 0)
            pltpu.sync_copy(rows_v, acc.at[sidx_v.at[j]], add=True)
            return carry

        lax.fori_loop(0, chs, chunk, 0)

        plsc.subcore_barrier()
        pltpu.sync_copy(acc.at[pl.ds(r0, RPS)], part_hbm.at[c, pl.ds(r0, RPS)])

        @pl.when(s == 0)
        def _out_rem():
            pltpu.sync_copy(acc.at[pl.ds(NS * RPS, REM)],
                            part_hbm.at[c, pl.ds(NS * RPS, REM)])

    return conv(nodes, filters, starts3, ends3)


def _combine_body(p1_ref, p2_ref, p3_ref, n_ref, o_ref):
    o_ref[...] = (n_ref[...]
                  + (p1_ref[0] + p1_ref[1])
                  + (p2_ref[0] + p2_ref[1])
                  + (p3_ref[0] + p3_ref[1]))


def _combine(parts, nodes):
    N, D = nodes.shape
    BN = 1000
    grid = N // BN
    pspec = pl.BlockSpec((2, BN, D), lambda i: (0, i, 0))
    return pl.pallas_call(
        _combine_body,
        grid=(grid,),
        in_specs=[pspec, pspec, pspec,
                  pl.BlockSpec((BN, D), lambda i: (i, 0))],
        out_specs=pl.BlockSpec((BN, D), lambda i: (i, 0)),
        out_shape=jax.ShapeDtypeStruct((N, D), jnp.float32),
    )(*parts, nodes)


def kernel(nodes, edges, edges_i, W1, b1, W2, b2):
    E = edges.shape[0]
    NW = 32
    align = NW * _K * 5
    segs = []
    acc_rows = 0
    for i, frac in enumerate(_SEGF):
        if i == len(_SEGF) - 1:
            nrows = E - acc_rows
        else:
            nrows = int(E * frac) // align * align
        segs.append((acc_rows, nrows))
        acc_rows += nrows
    assert acc_rows == E and all(r % align == 0 for _, r in segs)
    starts = edges_i[:, 0]
    ends = edges_i[:, 1]
    parts = []
    for row0, nrows in segs:
        f = _compute_filters_seg(edges, W1, b1, W2, b2, row0, nrows)
        chs = nrows // (NW * _K)
        s3 = lax.slice(starts, (row0,), (row0 + nrows,)).reshape(NW, chs, _K)
        e3 = lax.slice(ends, (row0,), (row0 + nrows,)).reshape(NW, chs, _K)
        parts.append(_sc_conv_seg(nodes, f, s3, e3))
    return _combine(parts, nodes)
